# Initial kernel scaffold; baseline (speedup 1.0000x reference)
import jax
import jax.numpy as jnp
from jax.experimental import pallas as pl


def kernel(global_data, segment_x, segment_edge_index, segment_edge_attr, segment_batch, dense_x, dense_edge_index, dense_batch, g0_W, g0_b, g0_gamma, g0_beta, g1_W, g1_b, g1_gamma, g1_beta, s0_Wl, s0_bl, s0_Wr, s0_br, s0_We, s0_be, s0_att, s0_bias, s0_gamma, s0_beta, s1_Wl, s1_bl, s1_Wr, s1_br, s1_We, s1_be, s1_att, s1_bias, s1_gamma, s1_beta, d0_Wl, d0_bl, d0_Wr, d0_br, d0_att, d0_bias, d0_gamma, d0_beta, d1_Wl, d1_bl, d1_Wr, d1_br, d1_att, d1_bias, d1_gamma, d1_beta, o_W, o_b):
    raise NotImplementedError("write your pallas kernel here")


# speedup vs baseline: 8.1010x; 8.1010x over previous
import jax
import jax.numpy as jnp
from jax import lax
from jax.experimental import pallas as pl
from jax.experimental.pallas import tpu as pltpu
from jax.experimental.pallas import tpu_sc as plsc

_N = 10000
_E = 320000
_B = 64
_H = 128
_ED = 16
_NC = 2
_NS = 16
_NW = _NC * _NS
_EPW = _E // _NW
_CH = 80
_NCHUNK = _EPW // _CH
_RPT = 640
_RPT_LAST = _N - 15 * _RPT
_SW = 16


def _sc_a_body(xl_hbm, xr_hbm, src_hbm, dst_hbm, v_out,
               src_v, dst_v, bufA, bufB, semA, semB):
    c = lax.axis_index("c")
    s = lax.axis_index("s")
    wid = c * _NS + s

    pltpu.sync_copy(src_hbm.at[wid], src_v)
    pltpu.sync_copy(dst_hbm.at[wid], dst_v)
    ebase = pl.multiple_of(wid * _EPW, 8)

    def chunk_body(ci, _):
        cpA = pltpu.async_copy(xl_hbm.at[src_v.at[ci]], bufA, semA)
        cpB = pltpu.async_copy(xr_hbm.at[dst_v.at[ci]], bufB, semB)
        cpA.wait()
        cpB.wait()

        def add_body(e, _):
            for u in range(8):
                bufA[e, pl.ds(u * 16, 16)] = (bufA[e, pl.ds(u * 16, 16)] +
                                              bufB[e, pl.ds(u * 16, 16)])
            return 0
        lax.fori_loop(0, _CH, add_body, 0)

        pltpu.sync_copy(bufA, v_out.at[pl.ds(ebase + ci * _CH, _CH)])
        return 0

    lax.fori_loop(0, _NCHUNK, chunk_body, 0)


def _sc_a(xl, xr, src_r, dst_r):
    return pl.kernel(
        _sc_a_body,
        out_type=jax.ShapeDtypeStruct((_E, _H), jnp.float32),
        mesh=plsc.VectorSubcoreMesh(core_axis_name="c", subcore_axis_name="s"),
        scratch_types=[
            pltpu.VMEM((_NCHUNK, _CH), jnp.int32),
            pltpu.VMEM((_NCHUNK, _CH), jnp.int32),
            pltpu.VMEM((_CH, _H), jnp.float32),
            pltpu.VMEM((_CH, _H), jnp.float32),
            pltpu.SemaphoreType.DMA,
            pltpu.SemaphoreType.DMA,
        ],
    )(xl, xr, src_r, dst_r)


def _sc_b_body(xl_hbm, w_hbm, src_hbm, dst_hbm, acc_out,
               src_v, dst_v, bufA, w_c, acc_sh, semA):
    c = lax.axis_index("c")
    s = lax.axis_index("s")
    wid = c * _NS + s
    tid = s

    z16f = jnp.zeros((16,), jnp.float32)
    lane = lax.iota(jnp.int32, 16)

    def _zrow(e, _):
        for u in range(8):
            bufA[e, pl.ds(u * 16, 16)] = z16f
        return 0
    lax.fori_loop(0, _CH, _zrow, 0)

    r0 = pl.multiple_of(tid * _RPT, 8)
    for j in range(5):
        pltpu.sync_copy(bufA, acc_sh.at[pl.ds(r0 + j * _CH, _CH)])

    @pl.when(tid < 15)
    def _():
        for j in range(5, 8):
            pltpu.sync_copy(bufA, acc_sh.at[pl.ds(r0 + j * _CH, _CH)])

    pltpu.sync_copy(src_hbm.at[wid], src_v)
    pltpu.sync_copy(dst_hbm.at[wid], dst_v)
    ebase = pl.multiple_of(wid * _EPW, 8)

    plsc.subcore_barrier()

    def chunk_body(ci, _):
        cpA = pltpu.async_copy(xl_hbm.at[src_v.at[ci]], bufA, semA)
        pltpu.sync_copy(w_hbm.at[pl.ds(ebase + ci * _CH, _CH)], w_c)
        cpA.wait()

        def group_body(g, _):
            wv = w_c[pl.ds(g * 16, 16)]
            for j in range(16):
                e = g * 16 + j
                w = wv[j]
                for u in range(8):
                    bufA[e, pl.ds(u * 16, 16)] = (
                        bufA[e, pl.ds(u * 16, 16)] * w)
            return 0
        lax.fori_loop(0, _CH // 16, group_body, 0)

        pltpu.sync_copy(bufA, acc_sh.at[dst_v.at[ci]], add=True)
        return 0

    lax.fori_loop(0, _NCHUNK, chunk_body, 0)

    plsc.subcore_barrier()

    def _out(nr):
        pltpu.sync_copy(acc_sh.at[pl.ds(r0, nr)], acc_out.at[c, pl.ds(r0, nr)])

    @pl.when(tid < 15)
    def _():
        _out(_RPT)

    @pl.when(tid == 15)
    def _():
        _out(_RPT_LAST)


def _sc_b(xl, w, src_r, dst_r):
    return pl.kernel(
        _sc_b_body,
        out_type=jax.ShapeDtypeStruct((_NC, _N, _H), jnp.float32),
        mesh=plsc.VectorSubcoreMesh(core_axis_name="c", subcore_axis_name="s"),
        scratch_types=[
            pltpu.VMEM((_NCHUNK, _CH), jnp.int32),
            pltpu.VMEM((_NCHUNK, _CH), jnp.int32),
            pltpu.VMEM((_CH, _H), jnp.float32),
            pltpu.VMEM((_CH,), jnp.float32),
            pltpu.VMEM_SHARED((_N, _H), jnp.float32),
            pltpu.SemaphoreType.DMA,
        ],
    )(xl, w, src_r, dst_r)


def _sc_c_body(w_hbm, dst_hbm, s_out,
               dst_v, w_c, s_tmp, s_sh):
    c = lax.axis_index("c")
    s = lax.axis_index("s")
    wid = c * _NS + s
    tid = s

    z16f = jnp.zeros((16,), jnp.float32)

    def _zw(i, _):
        w_c[pl.ds(i * 16, 16)] = z16f
        return 0
    lax.fori_loop(0, _CH // 16, _zw, 0)

    r0 = pl.multiple_of(tid * _RPT, 8)
    for j in range(5):
        pltpu.sync_copy(w_c, s_sh.at[pl.ds(r0 + j * _CH, _CH)])

    @pl.when(tid < 15)
    def _():
        for j in range(5, 8):
            pltpu.sync_copy(w_c, s_sh.at[pl.ds(r0 + j * _CH, _CH)])

    pltpu.sync_copy(dst_hbm.at[wid], dst_v)
    ebase = pl.multiple_of(wid * _EPW, 8)

    plsc.subcore_barrier()

    def chunk_body(ci, _):
        pltpu.sync_copy(w_hbm.at[pl.ds(ebase + ci * _CH, _CH)], w_c)
        pltpu.sync_copy(w_c, s_sh.at[dst_v.at[ci]], add=True)
        return 0

    lax.fori_loop(0, _NCHUNK, chunk_body, 0)

    plsc.subcore_barrier()

    def _out(nr):
        sob = pl.multiple_of(c * _N + r0, 8)
        pltpu.sync_copy(s_sh.at[pl.ds(r0, nr)], s_tmp.at[pl.ds(0, nr)])
        pltpu.sync_copy(s_tmp.at[pl.ds(0, nr)], s_out.at[pl.ds(sob, nr)])

    @pl.when(tid < 15)
    def _():
        _out(_RPT)

    @pl.when(tid == 15)
    def _():
        _out(_RPT_LAST)


def _sc_c(w, dst_r):
    return pl.kernel(
        _sc_c_body,
        out_type=jax.ShapeDtypeStruct((_NC * _N,), jnp.float32),
        mesh=plsc.VectorSubcoreMesh(core_axis_name="c", subcore_axis_name="s"),
        scratch_types=[
            pltpu.VMEM((_NCHUNK, _CH), jnp.int32),
            pltpu.VMEM((_CH,), jnp.float32),
            pltpu.VMEM((_RPT,), jnp.float32),
            pltpu.VMEM_SHARED((_N,), jnp.float32),
        ],
    )(w, dst_r)


_RB = 400
_NRB = _N // _RB
_EB = 4000
_NEB = _E // _EB


def _mm2_body(x_ref, wl_ref, bl_ref, wr_ref, br_ref, xl_ref, xr_ref):
    x = x_ref[...]
    xl_ref[...] = jnp.dot(x, wl_ref[...],
                          preferred_element_type=jnp.float32) + bl_ref[...]
    xr_ref[...] = jnp.dot(x, wr_ref[...],
                          preferred_element_type=jnp.float32) + br_ref[...]


def _mm2(x, wl, bl, wr, br):
    din = x.shape[1]
    return pl.pallas_call(
        _mm2_body,
        grid=(_NRB,),
        in_specs=[
            pl.BlockSpec((_RB, din), lambda i: (i, 0)),
            pl.BlockSpec((din, _H), lambda i: (0, 0)),
            pl.BlockSpec((1, _H), lambda i: (0, 0)),
            pl.BlockSpec((din, _H), lambda i: (0, 0)),
            pl.BlockSpec((1, _H), lambda i: (0, 0)),
        ],
        out_specs=[
            pl.BlockSpec((_RB, _H), lambda i: (i, 0)),
            pl.BlockSpec((_RB, _H), lambda i: (i, 0)),
        ],
        out_shape=[jax.ShapeDtypeStruct((_N, _H), jnp.float32)] * 2,
    )(x, wl, bl.reshape(1, _H), wr, br.reshape(1, _H))


def _logit_e_body(v_ref, ea_ref, we_ref, be_ref, att_ref, w_ref):
    m = (v_ref[...] +
         jnp.dot(ea_ref[...], we_ref[...],
                 preferred_element_type=jnp.float32) + be_ref[...])
    m = jnp.maximum(m, 0.2 * m)
    logit = jnp.sum(m * att_ref[...], axis=1)
    w_ref[...] = jnp.exp(logit).reshape(1, 1, _EB)


def _logit_e(v, ea, we, be, att):
    return pl.pallas_call(
        _logit_e_body,
        grid=(_NEB,),
        in_specs=[
            pl.BlockSpec((_EB, _H), lambda i: (i, 0)),
            pl.BlockSpec((_EB, _ED), lambda i: (i, 0)),
            pl.BlockSpec((_ED, _H), lambda i: (0, 0)),
            pl.BlockSpec((1, _H), lambda i: (0, 0)),
            pl.BlockSpec((1, _H), lambda i: (0, 0)),
        ],
        out_specs=pl.BlockSpec((1, 1, _EB), lambda i: (i, 0, 0)),
        out_shape=jax.ShapeDtypeStruct((_NEB, 1, _EB), jnp.float32),
    )(v, ea, we, be.reshape(1, _H), att.reshape(1, _H)).reshape(_E)


def _logit_d_body(v_ref, att_ref, w_ref):
    m = v_ref[...]
    m = jnp.maximum(m, 0.2 * m)
    logit = jnp.sum(m * att_ref[...], axis=1)
    w_ref[...] = jnp.exp(logit).reshape(1, 1, _EB)


def _logit_d(v, att):
    return pl.pallas_call(
        _logit_d_body,
        grid=(_NEB,),
        in_specs=[
            pl.BlockSpec((_EB, _H), lambda i: (i, 0)),
            pl.BlockSpec((1, _H), lambda i: (0, 0)),
        ],
        out_specs=pl.BlockSpec((1, 1, _EB), lambda i: (i, 0, 0)),
        out_shape=jax.ShapeDtypeStruct((_NEB, 1, _EB), jnp.float32),
    )(v, att.reshape(1, _H)).reshape(_E)


def _fin_a_body(acc_ref, s_ref, bias_ref, y_ref, stats_ref, st_sc):
    @pl.when(pl.program_id(0) == 0)
    def _():
        st_sc[...] = jnp.zeros_like(st_sc)

    sden = (s_ref[0, 0, 0, :] + s_ref[1, 0, 0, :] + 1e-16)[:, None]
    y = (acc_ref[0] + acc_ref[1]) / sden + bias_ref[...]
    y = jnp.maximum(y, 0.2 * y)
    y_ref[...] = y
    yr = y.reshape(_RB // 8, 8, _H)
    st_sc[0, :, :] = st_sc[0, :, :] + jnp.sum(yr, axis=0)
    st_sc[1, :, :] = st_sc[1, :, :] + jnp.sum(yr * yr, axis=0)
    stats_ref[...] = st_sc[...]


def _fin_a(acc, sden, bias):
    s4 = sden.reshape(_NC, _NRB, 1, _RB)
    return pl.pallas_call(
        _fin_a_body,
        grid=(_NRB,),
        in_specs=[
            pl.BlockSpec((_NC, _RB, _H), lambda i: (0, i, 0)),
            pl.BlockSpec((_NC, 1, 1, _RB), lambda i: (0, i, 0, 0)),
            pl.BlockSpec((1, _H), lambda i: (0, 0)),
        ],
        out_specs=[
            pl.BlockSpec((_RB, _H), lambda i: (i, 0)),
            pl.BlockSpec((2, 8, _H), lambda i: (0, 0, 0)),
        ],
        out_shape=[jax.ShapeDtypeStruct((_N, _H), jnp.float32),
                   jax.ShapeDtypeStruct((2, 8, _H), jnp.float32)],
        scratch_shapes=[pltpu.VMEM((2, 8, _H), jnp.float32)],
    )(acc, s4, bias.reshape(1, _H))


def _fin_b_body(y_ref, stats_ref, gamma_ref, beta_ref, x_ref):
    st = stats_ref[...]
    mean = jnp.sum(st[0], axis=0, keepdims=True) / _N
    var = jnp.sum(st[1], axis=0, keepdims=True) / _N - mean * mean
    y = y_ref[...]
    x_ref[...] = (gamma_ref[...] * (y - mean) / jnp.sqrt(var + 1e-5) +
                  beta_ref[...])


def _fin_b(y, stats, gamma, beta):
    return pl.pallas_call(
        _fin_b_body,
        grid=(_NRB,),
        in_specs=[
            pl.BlockSpec((_RB, _H), lambda i: (i, 0)),
            pl.BlockSpec((2, 8, _H), lambda i: (0, 0, 0)),
            pl.BlockSpec((1, _H), lambda i: (0, 0)),
            pl.BlockSpec((1, _H), lambda i: (0, 0)),
        ],
        out_specs=pl.BlockSpec((_RB, _H), lambda i: (i, 0)),
        out_shape=jax.ShapeDtypeStruct((_N, _H), jnp.float32),
    )(y, stats, gamma.reshape(1, _H), beta.reshape(1, _H))


def _g_body(g_ref, w0, b0, gm0, bt0, w1, b1, gm1, bt1, out_ref):
    g = g_ref[...]
    for w, b, gm, bt in ((w0, b0, gm0, bt0), (w1, b1, gm1, bt1)):
        g = jnp.dot(g, w[...], preferred_element_type=jnp.float32) + b[...]
        g = jnp.maximum(g, 0.2 * g)
        mean = jnp.mean(g, axis=0, keepdims=True)
        var = jnp.mean(g * g, axis=0, keepdims=True) - mean * mean
        g = gm[...] * (g - mean) / jnp.sqrt(var + 1e-5) + bt[...]
    out_ref[...] = g


def _g_path(g, w0, b0, gm0, bt0, w1, b1, gm1, bt1):
    r1 = lambda a: a.reshape(1, _H)
    return pl.pallas_call(
        _g_body,
        out_shape=jax.ShapeDtypeStruct((_B, _H), jnp.float32),
    )(g, w0, r1(b0), r1(gm0), r1(bt0), w1, r1(b1), r1(gm1), r1(bt1))


def _pool_body(sx_ref, dx_ref, sb_ref, db_ref, g_ref, ow_ref, ob_ref,
               out_ref, sp_sc, dp_sc):
    @pl.when(pl.program_id(0) == 0)
    def _():
        sp_sc[...] = jnp.full_like(sp_sc, -jnp.inf)
        dp_sc[...] = jnp.full_like(dp_sc, -jnp.inf)

    sx = sx_ref[...]
    dx = dx_ref[...]
    sb = sb_ref[0, 0, :][:, None]
    db = db_ref[0, 0, :][:, None]
    for b in range(_B):
        sm = jnp.max(jnp.where(sb == b, sx, -jnp.inf), axis=0)
        dm = jnp.max(jnp.where(db == b, dx, -jnp.inf), axis=0)
        sp_sc[b, :] = jnp.maximum(sp_sc[b, :], sm)
        dp_sc[b, :] = jnp.maximum(dp_sc[b, :], dm)

    @pl.when(pl.program_id(0) == _NRB - 1)
    def _():
        sp = sp_sc[...]
        dp = dp_sc[...]
        ow = ow_ref[...]
        acc = (jnp.sum(g_ref[...] * ow[0, 0:1, :], axis=1, keepdims=True) +
               jnp.sum(sp * ow[0, 1:2, :], axis=1, keepdims=True) +
               jnp.sum(dp * ow[0, 2:3, :], axis=1, keepdims=True))
        out_ref[...] = acc + ob_ref[...]


def _pool_out(sx, dx, sb3, db3, g, ow, ob):
    return pl.pallas_call(
        _pool_body,
        grid=(_NRB,),
        in_specs=[
            pl.BlockSpec((_RB, _H), lambda i: (i, 0)),
            pl.BlockSpec((_RB, _H), lambda i: (i, 0)),
            pl.BlockSpec((1, 1, _RB), lambda i: (i, 0, 0)),
            pl.BlockSpec((1, 1, _RB), lambda i: (i, 0, 0)),
            pl.BlockSpec((_B, _H), lambda i: (0, 0)),
            pl.BlockSpec((1, 3, _H), lambda i: (0, 0, 0)),
            pl.BlockSpec((1, 1), lambda i: (0, 0)),
        ],
        out_specs=pl.BlockSpec((_B, 1), lambda i: (0, 0)),
        out_shape=jax.ShapeDtypeStruct((_B, 1), jnp.float32),
        scratch_shapes=[pltpu.VMEM((_B, _H), jnp.float32),
                        pltpu.VMEM((_B, _H), jnp.float32)],
    )(sx, dx, sb3, db3, g, ow, ob.reshape(1, 1))


def _gat_layer(x, src_r, dst_r, wl, bl, wr, br, att, bias, gamma, beta,
               we=None, be=None, edge_attr=None):
    xl, xr = _mm2(x, wl, bl, wr, br)
    v = _sc_a(xl, xr, src_r, dst_r)
    if we is not None:
        w = _logit_e(v, edge_attr, we, be, att)
    else:
        w = _logit_d(v, att)
    acc = _sc_b(xl, w, src_r, dst_r)
    sden = _sc_c(w, dst_r)
    y, stats = _fin_a(acc, sden, bias)
    return _fin_b(y, stats, gamma, beta)


def kernel(global_data, segment_x, segment_edge_index, segment_edge_attr,
           segment_batch, dense_x, dense_edge_index, dense_batch,
           g0_W, g0_b, g0_gamma, g0_beta,
           g1_W, g1_b, g1_gamma, g1_beta,
           s0_Wl, s0_bl, s0_Wr, s0_br, s0_We, s0_be, s0_att, s0_bias,
           s0_gamma, s0_beta,
           s1_Wl, s1_bl, s1_Wr, s1_br, s1_We, s1_be, s1_att, s1_bias,
           s1_gamma, s1_beta,
           d0_Wl, d0_bl, d0_Wr, d0_br, d0_att, d0_bias, d0_gamma, d0_beta,
           d1_Wl, d1_bl, d1_Wr, d1_br, d1_att, d1_bias, d1_gamma, d1_beta,
           o_W, o_b):
    ssrc = segment_edge_index[0].astype(jnp.int32).reshape(_NW, _NCHUNK, _CH)
    sdst = segment_edge_index[1].astype(jnp.int32).reshape(_NW, _NCHUNK, _CH)
    dsrc = dense_edge_index[0].astype(jnp.int32).reshape(_NW, _NCHUNK, _CH)
    ddst = dense_edge_index[1].astype(jnp.int32).reshape(_NW, _NCHUNK, _CH)
    sb3 = segment_batch.astype(jnp.int32).reshape(_NRB, 1, _RB)
    db3 = dense_batch.astype(jnp.int32).reshape(_NRB, 1, _RB)

    g = _g_path(global_data, g0_W, g0_b, g0_gamma, g0_beta,
                g1_W, g1_b, g1_gamma, g1_beta)

    sx = segment_x
    sx = _gat_layer(sx, ssrc, sdst, s0_Wl, s0_bl, s0_Wr, s0_br, s0_att,
                    s0_bias, s0_gamma, s0_beta, s0_We, s0_be,
                    segment_edge_attr)
    sx = _gat_layer(sx, ssrc, sdst, s1_Wl, s1_bl, s1_Wr, s1_br, s1_att,
                    s1_bias, s1_gamma, s1_beta, s1_We, s1_be,
                    segment_edge_attr)

    dx = dense_x
    dx = _gat_layer(dx, dsrc, ddst, d0_Wl, d0_bl, d0_Wr, d0_br, d0_att,
                    d0_bias, d0_gamma, d0_beta)
    dx = _gat_layer(dx, dsrc, ddst, d1_Wl, d1_bl, d1_Wr, d1_br, d1_att,
                    d1_bias, d1_gamma, d1_beta)

    ow = jnp.transpose(o_W).reshape(1, 3, _H)
    return _pool_out(sx, dx, sb3, db3, g, ow, o_b)


# speedup vs baseline: 8.7548x; 1.0807x over previous
import jax
import jax.numpy as jnp
from jax import lax
from jax.experimental import pallas as pl
from jax.experimental.pallas import tpu as pltpu
from jax.experimental.pallas import tpu_sc as plsc

_N = 10000
_E = 320000
_B = 64
_H = 128
_ED = 16
_NC = 2
_NS = 16
_NW = _NC * _NS
_EPW = _E // _NW
_CH = 80
_NCHUNK = _EPW // _CH
_RPT = 640
_RPT_LAST = _N - 15 * _RPT
_SW = 16


def _sc_a_body(xl_hbm, xr_hbm, src_hbm, dst_hbm, v_out,
               src_v, dst_v, bufA, bufB, semA, semB):
    c = lax.axis_index("c")
    s = lax.axis_index("s")
    wid = c * _NS + s

    pltpu.sync_copy(src_hbm.at[wid], src_v)
    pltpu.sync_copy(dst_hbm.at[wid], dst_v)
    ebase = pl.multiple_of(wid * _EPW, 8)

    def chunk_body(ci, _):
        cpA = pltpu.async_copy(xl_hbm.at[src_v.at[ci]], bufA, semA)
        cpB = pltpu.async_copy(xr_hbm.at[dst_v.at[ci]], bufB, semB)
        cpA.wait()
        cpB.wait()

        def add_body(e, _):
            for u in range(8):
                bufA[e, pl.ds(u * 16, 16)] = (bufA[e, pl.ds(u * 16, 16)] +
                                              bufB[e, pl.ds(u * 16, 16)])
            return 0
        lax.fori_loop(0, _CH, add_body, 0)

        pltpu.sync_copy(bufA, v_out.at[pl.ds(ebase + ci * _CH, _CH)])
        return 0

    lax.fori_loop(0, _NCHUNK, chunk_body, 0)


def _sc_a(xl, xr, src_r, dst_r):
    return pl.kernel(
        _sc_a_body,
        out_type=jax.ShapeDtypeStruct((_E, _H), jnp.float32),
        mesh=plsc.VectorSubcoreMesh(core_axis_name="c", subcore_axis_name="s"),
        scratch_types=[
            pltpu.VMEM((_NCHUNK, _CH), jnp.int32),
            pltpu.VMEM((_NCHUNK, _CH), jnp.int32),
            pltpu.VMEM((_CH, _H), jnp.float32),
            pltpu.VMEM((_CH, _H), jnp.float32),
            pltpu.SemaphoreType.DMA,
            pltpu.SemaphoreType.DMA,
        ],
    )(xl, xr, src_r, dst_r)


def _sc_b_body(xl_hbm, w_hbm, src_hbm, dst_hbm, acc_out, s_out,
               src_v, dst_v, bufA, w_c, s_tmp, acc_sh, s_sh, semA):
    c = lax.axis_index("c")
    s = lax.axis_index("s")
    wid = c * _NS + s
    tid = s

    z16f = jnp.zeros((16,), jnp.float32)
    lane = lax.iota(jnp.int32, 16)

    def _zrow(e, _):
        for u in range(8):
            bufA[e, pl.ds(u * 16, 16)] = z16f
        return 0
    lax.fori_loop(0, _CH, _zrow, 0)

    def _zw(i, _):
        w_c[pl.ds(i * 16, 16)] = z16f
        return 0
    lax.fori_loop(0, _CH // 16, _zw, 0)

    r0 = pl.multiple_of(tid * _RPT, 8)
    for j in range(5):
        pltpu.sync_copy(bufA, acc_sh.at[pl.ds(r0 + j * _CH, _CH)])
        pltpu.sync_copy(w_c, s_sh.at[pl.ds(r0 + j * _CH, _CH)])

    @pl.when(tid < 15)
    def _():
        for j in range(5, 8):
            pltpu.sync_copy(bufA, acc_sh.at[pl.ds(r0 + j * _CH, _CH)])
            pltpu.sync_copy(w_c, s_sh.at[pl.ds(r0 + j * _CH, _CH)])

    pltpu.sync_copy(src_hbm.at[wid], src_v)
    pltpu.sync_copy(dst_hbm.at[wid], dst_v)
    ebase = pl.multiple_of(wid * _EPW, 8)

    plsc.subcore_barrier()

    def chunk_body(ci, _):
        cpA = pltpu.async_copy(xl_hbm.at[src_v.at[ci]], bufA, semA)
        pltpu.sync_copy(w_hbm.at[pl.ds(ebase + ci * _CH, _CH)], w_c)
        cpA.wait()

        def group_body(g, _):
            wv = w_c[pl.ds(g * 16, 16)]
            for j in range(16):
                e = g * 16 + j
                w = wv[j]
                for u in range(8):
                    bufA[e, pl.ds(u * 16, 16)] = (
                        bufA[e, pl.ds(u * 16, 16)] * w)
            return 0
        lax.fori_loop(0, _CH // 16, group_body, 0)

        pltpu.sync_copy(bufA, acc_sh.at[dst_v.at[ci]], add=True)
        pltpu.sync_copy(w_c, s_sh.at[dst_v.at[ci]], add=True)
        return 0

    lax.fori_loop(0, _NCHUNK, chunk_body, 0)

    plsc.subcore_barrier()

    def _out(nr):
        pltpu.sync_copy(acc_sh.at[pl.ds(r0, nr)], acc_out.at[c, pl.ds(r0, nr)])
        sob = pl.multiple_of(c * _N + r0, 8)
        pltpu.sync_copy(s_sh.at[pl.ds(r0, nr)], s_tmp.at[pl.ds(0, nr)])
        pltpu.sync_copy(s_tmp.at[pl.ds(0, nr)], s_out.at[pl.ds(sob, nr)])

    @pl.when(tid < 15)
    def _():
        _out(_RPT)

    @pl.when(tid == 15)
    def _():
        _out(_RPT_LAST)


def _sc_b(xl, w, src_r, dst_r):
    return pl.kernel(
        _sc_b_body,
        out_type=(jax.ShapeDtypeStruct((_NC, _N, _H), jnp.float32),
                  jax.ShapeDtypeStruct((_NC * _N,), jnp.float32)),
        mesh=plsc.VectorSubcoreMesh(core_axis_name="c", subcore_axis_name="s"),
        scratch_types=[
            pltpu.VMEM((_NCHUNK, _CH), jnp.int32),
            pltpu.VMEM((_NCHUNK, _CH), jnp.int32),
            pltpu.VMEM((_CH, _H), jnp.float32),
            pltpu.VMEM((_CH,), jnp.float32),
            pltpu.VMEM((_RPT,), jnp.float32),
            pltpu.VMEM_SHARED((_N, _H), jnp.float32),
            pltpu.VMEM_SHARED((_N,), jnp.float32),
            pltpu.SemaphoreType.DMA,
        ],
    )(xl, w, src_r, dst_r)


_RB = 400
_NRB = _N // _RB
_EB = 4000
_NEB = _E // _EB


def _mm2_body(x_ref, wl_ref, bl_ref, wr_ref, br_ref, xl_ref, xr_ref):
    x = x_ref[...]
    xl_ref[...] = jnp.dot(x, wl_ref[...],
                          preferred_element_type=jnp.float32) + bl_ref[...]
    xr_ref[...] = jnp.dot(x, wr_ref[...],
                          preferred_element_type=jnp.float32) + br_ref[...]


def _mm2(x, wl, bl, wr, br):
    din = x.shape[1]
    return pl.pallas_call(
        _mm2_body,
        grid=(_NRB,),
        in_specs=[
            pl.BlockSpec((_RB, din), lambda i: (i, 0)),
            pl.BlockSpec((din, _H), lambda i: (0, 0)),
            pl.BlockSpec((1, _H), lambda i: (0, 0)),
            pl.BlockSpec((din, _H), lambda i: (0, 0)),
            pl.BlockSpec((1, _H), lambda i: (0, 0)),
        ],
        out_specs=[
            pl.BlockSpec((_RB, _H), lambda i: (i, 0)),
            pl.BlockSpec((_RB, _H), lambda i: (i, 0)),
        ],
        out_shape=[jax.ShapeDtypeStruct((_N, _H), jnp.float32)] * 2,
    )(x, wl, bl.reshape(1, _H), wr, br.reshape(1, _H))


def _logit_e_body(v_ref, ea_ref, we_ref, be_ref, att_ref, w_ref):
    m = (v_ref[...] +
         jnp.dot(ea_ref[...], we_ref[...],
                 preferred_element_type=jnp.float32) + be_ref[...])
    m = jnp.maximum(m, 0.2 * m)
    logit = jnp.sum(m * att_ref[...], axis=1)
    w_ref[...] = jnp.exp(logit).reshape(1, 1, _EB)


def _logit_e(v, ea, we, be, att):
    return pl.pallas_call(
        _logit_e_body,
        grid=(_NEB,),
        in_specs=[
            pl.BlockSpec((_EB, _H), lambda i: (i, 0)),
            pl.BlockSpec((_EB, _ED), lambda i: (i, 0)),
            pl.BlockSpec((_ED, _H), lambda i: (0, 0)),
            pl.BlockSpec((1, _H), lambda i: (0, 0)),
            pl.BlockSpec((1, _H), lambda i: (0, 0)),
        ],
        out_specs=pl.BlockSpec((1, 1, _EB), lambda i: (i, 0, 0)),
        out_shape=jax.ShapeDtypeStruct((_NEB, 1, _EB), jnp.float32),
    )(v, ea, we, be.reshape(1, _H), att.reshape(1, _H)).reshape(_E)


def _logit_d_body(v_ref, att_ref, w_ref):
    m = v_ref[...]
    m = jnp.maximum(m, 0.2 * m)
    logit = jnp.sum(m * att_ref[...], axis=1)
    w_ref[...] = jnp.exp(logit).reshape(1, 1, _EB)


def _logit_d(v, att):
    return pl.pallas_call(
        _logit_d_body,
        grid=(_NEB,),
        in_specs=[
            pl.BlockSpec((_EB, _H), lambda i: (i, 0)),
            pl.BlockSpec((1, _H), lambda i: (0, 0)),
        ],
        out_specs=pl.BlockSpec((1, 1, _EB), lambda i: (i, 0, 0)),
        out_shape=jax.ShapeDtypeStruct((_NEB, 1, _EB), jnp.float32),
    )(v, att.reshape(1, _H)).reshape(_E)


def _fin_a_body(acc_ref, s_ref, bias_ref, y_ref, stats_ref, st_sc):
    @pl.when(pl.program_id(0) == 0)
    def _():
        st_sc[...] = jnp.zeros_like(st_sc)

    sden = (s_ref[0, 0, 0, :] + s_ref[1, 0, 0, :] + 1e-16)[:, None]
    y = (acc_ref[0] + acc_ref[1]) / sden + bias_ref[...]
    y = jnp.maximum(y, 0.2 * y)
    y_ref[...] = y
    yr = y.reshape(_RB // 8, 8, _H)
    st_sc[0, :, :] = st_sc[0, :, :] + jnp.sum(yr, axis=0)
    st_sc[1, :, :] = st_sc[1, :, :] + jnp.sum(yr * yr, axis=0)
    stats_ref[...] = st_sc[...]


def _fin_a(acc, sden, bias):
    s4 = sden.reshape(_NC, _NRB, 1, _RB)
    return pl.pallas_call(
        _fin_a_body,
        grid=(_NRB,),
        in_specs=[
            pl.BlockSpec((_NC, _RB, _H), lambda i: (0, i, 0)),
            pl.BlockSpec((_NC, 1, 1, _RB), lambda i: (0, i, 0, 0)),
            pl.BlockSpec((1, _H), lambda i: (0, 0)),
        ],
        out_specs=[
            pl.BlockSpec((_RB, _H), lambda i: (i, 0)),
            pl.BlockSpec((2, 8, _H), lambda i: (0, 0, 0)),
        ],
        out_shape=[jax.ShapeDtypeStruct((_N, _H), jnp.float32),
                   jax.ShapeDtypeStruct((2, 8, _H), jnp.float32)],
        scratch_shapes=[pltpu.VMEM((2, 8, _H), jnp.float32)],
    )(acc, s4, bias.reshape(1, _H))


def _fin_b_body(y_ref, stats_ref, gamma_ref, beta_ref, x_ref):
    st = stats_ref[...]
    mean = jnp.sum(st[0], axis=0, keepdims=True) / _N
    var = jnp.sum(st[1], axis=0, keepdims=True) / _N - mean * mean
    y = y_ref[...]
    x_ref[...] = (gamma_ref[...] * (y - mean) / jnp.sqrt(var + 1e-5) +
                  beta_ref[...])


def _fin_b(y, stats, gamma, beta):
    return pl.pallas_call(
        _fin_b_body,
        grid=(_NRB,),
        in_specs=[
            pl.BlockSpec((_RB, _H), lambda i: (i, 0)),
            pl.BlockSpec((2, 8, _H), lambda i: (0, 0, 0)),
            pl.BlockSpec((1, _H), lambda i: (0, 0)),
            pl.BlockSpec((1, _H), lambda i: (0, 0)),
        ],
        out_specs=pl.BlockSpec((_RB, _H), lambda i: (i, 0)),
        out_shape=jax.ShapeDtypeStruct((_N, _H), jnp.float32),
    )(y, stats, gamma.reshape(1, _H), beta.reshape(1, _H))


def _g_body(g_ref, w0, b0, gm0, bt0, w1, b1, gm1, bt1, out_ref):
    g = g_ref[...]
    for w, b, gm, bt in ((w0, b0, gm0, bt0), (w1, b1, gm1, bt1)):
        g = jnp.dot(g, w[...], preferred_element_type=jnp.float32) + b[...]
        g = jnp.maximum(g, 0.2 * g)
        mean = jnp.mean(g, axis=0, keepdims=True)
        var = jnp.mean(g * g, axis=0, keepdims=True) - mean * mean
        g = gm[...] * (g - mean) / jnp.sqrt(var + 1e-5) + bt[...]
    out_ref[...] = g


def _g_path(g, w0, b0, gm0, bt0, w1, b1, gm1, bt1):
    r1 = lambda a: a.reshape(1, _H)
    return pl.pallas_call(
        _g_body,
        out_shape=jax.ShapeDtypeStruct((_B, _H), jnp.float32),
    )(g, w0, r1(b0), r1(gm0), r1(bt0), w1, r1(b1), r1(gm1), r1(bt1))


def _pool_body(sx_ref, dx_ref, sb_ref, db_ref, g_ref, ow_ref, ob_ref,
               out_ref, sp_sc, dp_sc):
    @pl.when(pl.program_id(0) == 0)
    def _():
        sp_sc[...] = jnp.full_like(sp_sc, -jnp.inf)
        dp_sc[...] = jnp.full_like(dp_sc, -jnp.inf)

    sx = sx_ref[...]
    dx = dx_ref[...]
    sb = sb_ref[0, 0, :][:, None]
    db = db_ref[0, 0, :][:, None]
    for b in range(_B):
        sm = jnp.max(jnp.where(sb == b, sx, -jnp.inf), axis=0)
        dm = jnp.max(jnp.where(db == b, dx, -jnp.inf), axis=0)
        sp_sc[b, :] = jnp.maximum(sp_sc[b, :], sm)
        dp_sc[b, :] = jnp.maximum(dp_sc[b, :], dm)

    @pl.when(pl.program_id(0) == _NRB - 1)
    def _():
        sp = sp_sc[...]
        dp = dp_sc[...]
        ow = ow_ref[...]
        acc = (jnp.sum(g_ref[...] * ow[0, 0:1, :], axis=1, keepdims=True) +
               jnp.sum(sp * ow[0, 1:2, :], axis=1, keepdims=True) +
               jnp.sum(dp * ow[0, 2:3, :], axis=1, keepdims=True))
        out_ref[...] = acc + ob_ref[...]


def _pool_out(sx, dx, sb3, db3, g, ow, ob):
    return pl.pallas_call(
        _pool_body,
        grid=(_NRB,),
        in_specs=[
            pl.BlockSpec((_RB, _H), lambda i: (i, 0)),
            pl.BlockSpec((_RB, _H), lambda i: (i, 0)),
            pl.BlockSpec((1, 1, _RB), lambda i: (i, 0, 0)),
            pl.BlockSpec((1, 1, _RB), lambda i: (i, 0, 0)),
            pl.BlockSpec((_B, _H), lambda i: (0, 0)),
            pl.BlockSpec((1, 3, _H), lambda i: (0, 0, 0)),
            pl.BlockSpec((1, 1), lambda i: (0, 0)),
        ],
        out_specs=pl.BlockSpec((_B, 1), lambda i: (0, 0)),
        out_shape=jax.ShapeDtypeStruct((_B, 1), jnp.float32),
        scratch_shapes=[pltpu.VMEM((_B, _H), jnp.float32),
                        pltpu.VMEM((_B, _H), jnp.float32)],
    )(sx, dx, sb3, db3, g, ow, ob.reshape(1, 1))


def _gat_layer(x, src_r, dst_r, wl, bl, wr, br, att, bias, gamma, beta,
               we=None, be=None, edge_attr=None):
    xl, xr = _mm2(x, wl, bl, wr, br)
    v = _sc_a(xl, xr, src_r, dst_r)
    if we is not None:
        w = _logit_e(v, edge_attr, we, be, att)
    else:
        w = _logit_d(v, att)
    acc, sden = _sc_b(xl, w, src_r, dst_r)
    y, stats = _fin_a(acc, sden, bias)
    return _fin_b(y, stats, gamma, beta)


def kernel(global_data, segment_x, segment_edge_index, segment_edge_attr,
           segment_batch, dense_x, dense_edge_index, dense_batch,
           g0_W, g0_b, g0_gamma, g0_beta,
           g1_W, g1_b, g1_gamma, g1_beta,
           s0_Wl, s0_bl, s0_Wr, s0_br, s0_We, s0_be, s0_att, s0_bias,
           s0_gamma, s0_beta,
           s1_Wl, s1_bl, s1_Wr, s1_br, s1_We, s1_be, s1_att, s1_bias,
           s1_gamma, s1_beta,
           d0_Wl, d0_bl, d0_Wr, d0_br, d0_att, d0_bias, d0_gamma, d0_beta,
           d1_Wl, d1_bl, d1_Wr, d1_br, d1_att, d1_bias, d1_gamma, d1_beta,
           o_W, o_b):
    ssrc = segment_edge_index[0].astype(jnp.int32).reshape(_NW, _NCHUNK, _CH)
    sdst = segment_edge_index[1].astype(jnp.int32).reshape(_NW, _NCHUNK, _CH)
    dsrc = dense_edge_index[0].astype(jnp.int32).reshape(_NW, _NCHUNK, _CH)
    ddst = dense_edge_index[1].astype(jnp.int32).reshape(_NW, _NCHUNK, _CH)
    sb3 = segment_batch.astype(jnp.int32).reshape(_NRB, 1, _RB)
    db3 = dense_batch.astype(jnp.int32).reshape(_NRB, 1, _RB)

    g = _g_path(global_data, g0_W, g0_b, g0_gamma, g0_beta,
                g1_W, g1_b, g1_gamma, g1_beta)

    sx = segment_x
    sx = _gat_layer(sx, ssrc, sdst, s0_Wl, s0_bl, s0_Wr, s0_br, s0_att,
                    s0_bias, s0_gamma, s0_beta, s0_We, s0_be,
                    segment_edge_attr)
    sx = _gat_layer(sx, ssrc, sdst, s1_Wl, s1_bl, s1_Wr, s1_br, s1_att,
                    s1_bias, s1_gamma, s1_beta, s1_We, s1_be,
                    segment_edge_attr)

    dx = dense_x
    dx = _gat_layer(dx, dsrc, ddst, d0_Wl, d0_bl, d0_Wr, d0_br, d0_att,
                    d0_bias, d0_gamma, d0_beta)
    dx = _gat_layer(dx, dsrc, ddst, d1_Wl, d1_bl, d1_Wr, d1_br, d1_att,
                    d1_bias, d1_gamma, d1_beta)

    ow = jnp.transpose(o_W).reshape(1, 3, _H)
    return _pool_out(sx, dx, sb3, db3, g, ow, o_b)


# speedup vs baseline: 10.2910x; 1.1755x over previous
import jax
import jax.numpy as jnp
from jax import lax
from jax.experimental import pallas as pl
from jax.experimental.pallas import tpu as pltpu
from jax.experimental.pallas import tpu_sc as plsc

_N = 10000
_E = 320000
_B = 64
_H = 128
_ED = 16
_NC = 2
_NS = 16
_NW = _NC * _NS
_EPW = _E // _NW
_CH = 80
_NCHUNK = _EPW // _CH
_RPT = 640
_RPT_LAST = _N - 15 * _RPT
_SW = 16


def _sc_a_body(xl_hbm, xr_hbm, src_hbm, dst_hbm, v_out,
               src_v, dst_v, bufA0, bufB0, out0, bufA1, bufB1, out1,
               semA0, semB0, semO0, semA1, semB1, semO1):
    c = lax.axis_index("c")
    s = lax.axis_index("s")
    wid = c * _NS + s

    pltpu.sync_copy(src_hbm.at[wid], src_v)
    pltpu.sync_copy(dst_hbm.at[wid], dst_v)
    ebase = pl.multiple_of(wid * _EPW, 8)

    bufs = ((bufA0, bufB0, out0, semA0, semB0, semO0),
            (bufA1, bufB1, out1, semA1, semB1, semO1))

    def pre(ci, slot):
        bufA, bufB, _, sA, sB, _ = bufs[slot]
        pltpu.async_copy(xl_hbm.at[src_v.at[ci]], bufA, sA)
        pltpu.async_copy(xr_hbm.at[dst_v.at[ci]], bufB, sB)

    def proc(ci, slot):
        bufA, bufB, out, sA, sB, sO = bufs[slot]
        pltpu.make_async_copy(xl_hbm.at[src_v.at[ci]], bufA, sA).wait()
        pltpu.make_async_copy(xr_hbm.at[dst_v.at[ci]], bufB, sB).wait()

        @pl.when(ci >= 2)
        def _():
            pltpu.make_async_copy(
                out, v_out.at[pl.ds(ebase + (ci - 2) * _CH, _CH)], sO).wait()

        def add_body(e, _):
            for u in range(8):
                out[e, pl.ds(u * 16, 16)] = (bufA[e, pl.ds(u * 16, 16)] +
                                             bufB[e, pl.ds(u * 16, 16)])
            return 0
        lax.fori_loop(0, _CH, add_body, 0)

        pltpu.async_copy(out, v_out.at[pl.ds(ebase + ci * _CH, _CH)], sO)

    pre(0, 0)
    pre(1, 1)

    def body2(i, _):
        ci0 = 2 * i
        proc(ci0, 0)
        pre(ci0 + 2, 0)
        proc(ci0 + 1, 1)

        @pl.when(ci0 + 3 < _NCHUNK)
        def _():
            pre(ci0 + 3, 1)
        return 0

    lax.fori_loop(0, (_NCHUNK - 1) // 2, body2, 0)
    proc(_NCHUNK - 1, 0)

    pltpu.make_async_copy(
        out1, v_out.at[pl.ds(ebase + (_NCHUNK - 2) * _CH, _CH)], semO1).wait()
    pltpu.make_async_copy(
        out0, v_out.at[pl.ds(ebase + (_NCHUNK - 1) * _CH, _CH)], semO0).wait()


def _sc_a(xl, xr, src_r, dst_r):
    return pl.kernel(
        _sc_a_body,
        out_type=jax.ShapeDtypeStruct((_E, _H), jnp.float32),
        mesh=plsc.VectorSubcoreMesh(core_axis_name="c", subcore_axis_name="s"),
        scratch_types=[
            pltpu.VMEM((_NCHUNK, _CH), jnp.int32),
            pltpu.VMEM((_NCHUNK, _CH), jnp.int32),
        ] + [pltpu.VMEM((_CH, _H), jnp.float32)] * 6 + [
            pltpu.SemaphoreType.DMA,
        ] * 6,
    )(xl, xr, src_r, dst_r)


def _sc_b_body(xl_hbm, w_hbm, src_hbm, dst_hbm, acc_out, s_out,
               src_v, dst_v, bufA, w_c, s_tmp, acc_sh, s_sh, semA):
    c = lax.axis_index("c")
    s = lax.axis_index("s")
    wid = c * _NS + s
    tid = s

    z16f = jnp.zeros((16,), jnp.float32)
    lane = lax.iota(jnp.int32, 16)

    def _zrow(e, _):
        for u in range(8):
            bufA[e, pl.ds(u * 16, 16)] = z16f
        return 0
    lax.fori_loop(0, _CH, _zrow, 0)

    def _zw(i, _):
        w_c[pl.ds(i * 16, 16)] = z16f
        return 0
    lax.fori_loop(0, _CH // 16, _zw, 0)

    r0 = pl.multiple_of(tid * _RPT, 8)
    for j in range(5):
        pltpu.sync_copy(bufA, acc_sh.at[pl.ds(r0 + j * _CH, _CH)])
        pltpu.sync_copy(w_c, s_sh.at[pl.ds(r0 + j * _CH, _CH)])

    @pl.when(tid < 15)
    def _():
        for j in range(5, 8):
            pltpu.sync_copy(bufA, acc_sh.at[pl.ds(r0 + j * _CH, _CH)])
            pltpu.sync_copy(w_c, s_sh.at[pl.ds(r0 + j * _CH, _CH)])

    pltpu.sync_copy(src_hbm.at[wid], src_v)
    pltpu.sync_copy(dst_hbm.at[wid], dst_v)
    ebase = pl.multiple_of(wid * _EPW, 8)

    plsc.subcore_barrier()

    def chunk_body(ci, _):
        cpA = pltpu.async_copy(xl_hbm.at[src_v.at[ci]], bufA, semA)
        pltpu.sync_copy(w_hbm.at[pl.ds(ebase + ci * _CH, _CH)], w_c)
        cpA.wait()

        def group_body(g, _):
            wv = w_c[pl.ds(g * 16, 16)]
            for j in range(16):
                e = g * 16 + j
                w = wv[j]
                for u in range(8):
                    bufA[e, pl.ds(u * 16, 16)] = (
                        bufA[e, pl.ds(u * 16, 16)] * w)
            return 0
        lax.fori_loop(0, _CH // 16, group_body, 0)

        pltpu.sync_copy(bufA, acc_sh.at[dst_v.at[ci]], add=True)
        pltpu.sync_copy(w_c, s_sh.at[dst_v.at[ci]], add=True)
        return 0

    lax.fori_loop(0, _NCHUNK, chunk_body, 0)

    plsc.subcore_barrier()

    def _out(nr):
        pltpu.sync_copy(acc_sh.at[pl.ds(r0, nr)], acc_out.at[c, pl.ds(r0, nr)])
        sob = pl.multiple_of(c * _N + r0, 8)
        pltpu.sync_copy(s_sh.at[pl.ds(r0, nr)], s_tmp.at[pl.ds(0, nr)])
        pltpu.sync_copy(s_tmp.at[pl.ds(0, nr)], s_out.at[pl.ds(sob, nr)])

    @pl.when(tid < 15)
    def _():
        _out(_RPT)

    @pl.when(tid == 15)
    def _():
        _out(_RPT_LAST)


def _sc_b(xl, w, src_r, dst_r):
    return pl.kernel(
        _sc_b_body,
        out_type=(jax.ShapeDtypeStruct((_NC, _N, _H), jnp.float32),
                  jax.ShapeDtypeStruct((_NC * _N,), jnp.float32)),
        mesh=plsc.VectorSubcoreMesh(core_axis_name="c", subcore_axis_name="s"),
        scratch_types=[
            pltpu.VMEM((_NCHUNK, _CH), jnp.int32),
            pltpu.VMEM((_NCHUNK, _CH), jnp.int32),
            pltpu.VMEM((_CH, _H), jnp.float32),
            pltpu.VMEM((_CH,), jnp.float32),
            pltpu.VMEM((_RPT,), jnp.float32),
            pltpu.VMEM_SHARED((_N, _H), jnp.float32),
            pltpu.VMEM_SHARED((_N,), jnp.float32),
            pltpu.SemaphoreType.DMA,
        ],
    )(xl, w, src_r, dst_r)


_RB = 400
_NRB = _N // _RB
_EB = 4000
_NEB = _E // _EB


def _mm2_body(x_ref, wl_ref, bl_ref, wr_ref, br_ref, xl_ref, xr_ref):
    x = x_ref[...]
    xl_ref[...] = jnp.dot(x, wl_ref[...],
                          preferred_element_type=jnp.float32) + bl_ref[...]
    xr_ref[...] = jnp.dot(x, wr_ref[...],
                          preferred_element_type=jnp.float32) + br_ref[...]


def _mm2(x, wl, bl, wr, br):
    din = x.shape[1]
    return pl.pallas_call(
        _mm2_body,
        grid=(_NRB,),
        in_specs=[
            pl.BlockSpec((_RB, din), lambda i: (i, 0)),
            pl.BlockSpec((din, _H), lambda i: (0, 0)),
            pl.BlockSpec((1, _H), lambda i: (0, 0)),
            pl.BlockSpec((din, _H), lambda i: (0, 0)),
            pl.BlockSpec((1, _H), lambda i: (0, 0)),
        ],
        out_specs=[
            pl.BlockSpec((_RB, _H), lambda i: (i, 0)),
            pl.BlockSpec((_RB, _H), lambda i: (i, 0)),
        ],
        out_shape=[jax.ShapeDtypeStruct((_N, _H), jnp.float32)] * 2,
    )(x, wl, bl.reshape(1, _H), wr, br.reshape(1, _H))


def _logit_e_body(v_ref, ea_ref, we_ref, be_ref, att_ref, w_ref):
    m = (v_ref[...] +
         jnp.dot(ea_ref[...], we_ref[...],
                 preferred_element_type=jnp.float32) + be_ref[...])
    m = jnp.maximum(m, 0.2 * m)
    logit = jnp.sum(m * att_ref[...], axis=1)
    w_ref[...] = jnp.exp(logit).reshape(1, 1, _EB)


def _logit_e(v, ea, we, be, att):
    return pl.pallas_call(
        _logit_e_body,
        grid=(_NEB,),
        in_specs=[
            pl.BlockSpec((_EB, _H), lambda i: (i, 0)),
            pl.BlockSpec((_EB, _ED), lambda i: (i, 0)),
            pl.BlockSpec((_ED, _H), lambda i: (0, 0)),
            pl.BlockSpec((1, _H), lambda i: (0, 0)),
            pl.BlockSpec((1, _H), lambda i: (0, 0)),
        ],
        out_specs=pl.BlockSpec((1, 1, _EB), lambda i: (i, 0, 0)),
        out_shape=jax.ShapeDtypeStruct((_NEB, 1, _EB), jnp.float32),
    )(v, ea, we, be.reshape(1, _H), att.reshape(1, _H)).reshape(_E)


def _logit_d_body(v_ref, att_ref, w_ref):
    m = v_ref[...]
    m = jnp.maximum(m, 0.2 * m)
    logit = jnp.sum(m * att_ref[...], axis=1)
    w_ref[...] = jnp.exp(logit).reshape(1, 1, _EB)


def _logit_d(v, att):
    return pl.pallas_call(
        _logit_d_body,
        grid=(_NEB,),
        in_specs=[
            pl.BlockSpec((_EB, _H), lambda i: (i, 0)),
            pl.BlockSpec((1, _H), lambda i: (0, 0)),
        ],
        out_specs=pl.BlockSpec((1, 1, _EB), lambda i: (i, 0, 0)),
        out_shape=jax.ShapeDtypeStruct((_NEB, 1, _EB), jnp.float32),
    )(v, att.reshape(1, _H)).reshape(_E)


def _fin_a_body(acc_ref, s_ref, bias_ref, y_ref, stats_ref, st_sc):
    @pl.when(pl.program_id(0) == 0)
    def _():
        st_sc[...] = jnp.zeros_like(st_sc)

    sden = (s_ref[0, 0, 0, :] + s_ref[1, 0, 0, :] + 1e-16)[:, None]
    y = (acc_ref[0] + acc_ref[1]) / sden + bias_ref[...]
    y = jnp.maximum(y, 0.2 * y)
    y_ref[...] = y
    yr = y.reshape(_RB // 8, 8, _H)
    st_sc[0, :, :] = st_sc[0, :, :] + jnp.sum(yr, axis=0)
    st_sc[1, :, :] = st_sc[1, :, :] + jnp.sum(yr * yr, axis=0)
    stats_ref[...] = st_sc[...]


def _fin_a(acc, sden, bias):
    s4 = sden.reshape(_NC, _NRB, 1, _RB)
    return pl.pallas_call(
        _fin_a_body,
        grid=(_NRB,),
        in_specs=[
            pl.BlockSpec((_NC, _RB, _H), lambda i: (0, i, 0)),
            pl.BlockSpec((_NC, 1, 1, _RB), lambda i: (0, i, 0, 0)),
            pl.BlockSpec((1, _H), lambda i: (0, 0)),
        ],
        out_specs=[
            pl.BlockSpec((_RB, _H), lambda i: (i, 0)),
            pl.BlockSpec((2, 8, _H), lambda i: (0, 0, 0)),
        ],
        out_shape=[jax.ShapeDtypeStruct((_N, _H), jnp.float32),
                   jax.ShapeDtypeStruct((2, 8, _H), jnp.float32)],
        scratch_shapes=[pltpu.VMEM((2, 8, _H), jnp.float32)],
    )(acc, s4, bias.reshape(1, _H))


def _fin_b_body(y_ref, stats_ref, gamma_ref, beta_ref, x_ref):
    st = stats_ref[...]
    mean = jnp.sum(st[0], axis=0, keepdims=True) / _N
    var = jnp.sum(st[1], axis=0, keepdims=True) / _N - mean * mean
    y = y_ref[...]
    x_ref[...] = (gamma_ref[...] * (y - mean) / jnp.sqrt(var + 1e-5) +
                  beta_ref[...])


def _fin_b(y, stats, gamma, beta):
    return pl.pallas_call(
        _fin_b_body,
        grid=(_NRB,),
        in_specs=[
            pl.BlockSpec((_RB, _H), lambda i: (i, 0)),
            pl.BlockSpec((2, 8, _H), lambda i: (0, 0, 0)),
            pl.BlockSpec((1, _H), lambda i: (0, 0)),
            pl.BlockSpec((1, _H), lambda i: (0, 0)),
        ],
        out_specs=pl.BlockSpec((_RB, _H), lambda i: (i, 0)),
        out_shape=jax.ShapeDtypeStruct((_N, _H), jnp.float32),
    )(y, stats, gamma.reshape(1, _H), beta.reshape(1, _H))


def _g_body(g_ref, w0, b0, gm0, bt0, w1, b1, gm1, bt1, out_ref):
    g = g_ref[...]
    for w, b, gm, bt in ((w0, b0, gm0, bt0), (w1, b1, gm1, bt1)):
        g = jnp.dot(g, w[...], preferred_element_type=jnp.float32) + b[...]
        g = jnp.maximum(g, 0.2 * g)
        mean = jnp.mean(g, axis=0, keepdims=True)
        var = jnp.mean(g * g, axis=0, keepdims=True) - mean * mean
        g = gm[...] * (g - mean) / jnp.sqrt(var + 1e-5) + bt[...]
    out_ref[...] = g


def _g_path(g, w0, b0, gm0, bt0, w1, b1, gm1, bt1):
    r1 = lambda a: a.reshape(1, _H)
    return pl.pallas_call(
        _g_body,
        out_shape=jax.ShapeDtypeStruct((_B, _H), jnp.float32),
    )(g, w0, r1(b0), r1(gm0), r1(bt0), w1, r1(b1), r1(gm1), r1(bt1))


def _pool_body(sx_ref, dx_ref, sb_ref, db_ref, g_ref, ow_ref, ob_ref,
               out_ref, sp_sc, dp_sc):
    @pl.when(pl.program_id(0) == 0)
    def _():
        sp_sc[...] = jnp.full_like(sp_sc, -jnp.inf)
        dp_sc[...] = jnp.full_like(dp_sc, -jnp.inf)

    sx = sx_ref[...]
    dx = dx_ref[...]
    sb = sb_ref[0, 0, :][:, None]
    db = db_ref[0, 0, :][:, None]
    for b in range(_B):
        sm = jnp.max(jnp.where(sb == b, sx, -jnp.inf), axis=0)
        dm = jnp.max(jnp.where(db == b, dx, -jnp.inf), axis=0)
        sp_sc[b, :] = jnp.maximum(sp_sc[b, :], sm)
        dp_sc[b, :] = jnp.maximum(dp_sc[b, :], dm)

    @pl.when(pl.program_id(0) == _NRB - 1)
    def _():
        sp = sp_sc[...]
        dp = dp_sc[...]
        ow = ow_ref[...]
        acc = (jnp.sum(g_ref[...] * ow[0, 0:1, :], axis=1, keepdims=True) +
               jnp.sum(sp * ow[0, 1:2, :], axis=1, keepdims=True) +
               jnp.sum(dp * ow[0, 2:3, :], axis=1, keepdims=True))
        out_ref[...] = acc + ob_ref[...]


def _pool_out(sx, dx, sb3, db3, g, ow, ob):
    return pl.pallas_call(
        _pool_body,
        grid=(_NRB,),
        in_specs=[
            pl.BlockSpec((_RB, _H), lambda i: (i, 0)),
            pl.BlockSpec((_RB, _H), lambda i: (i, 0)),
            pl.BlockSpec((1, 1, _RB), lambda i: (i, 0, 0)),
            pl.BlockSpec((1, 1, _RB), lambda i: (i, 0, 0)),
            pl.BlockSpec((_B, _H), lambda i: (0, 0)),
            pl.BlockSpec((1, 3, _H), lambda i: (0, 0, 0)),
            pl.BlockSpec((1, 1), lambda i: (0, 0)),
        ],
        out_specs=pl.BlockSpec((_B, 1), lambda i: (0, 0)),
        out_shape=jax.ShapeDtypeStruct((_B, 1), jnp.float32),
        scratch_shapes=[pltpu.VMEM((_B, _H), jnp.float32),
                        pltpu.VMEM((_B, _H), jnp.float32)],
    )(sx, dx, sb3, db3, g, ow, ob.reshape(1, 1))


def _gat_layer(x, src_r, dst_r, wl, bl, wr, br, att, bias, gamma, beta,
               we=None, be=None, edge_attr=None):
    xl, xr = _mm2(x, wl, bl, wr, br)
    v = _sc_a(xl, xr, src_r, dst_r)
    if we is not None:
        w = _logit_e(v, edge_attr, we, be, att)
    else:
        w = _logit_d(v, att)
    acc, sden = _sc_b(xl, w, src_r, dst_r)
    y, stats = _fin_a(acc, sden, bias)
    return _fin_b(y, stats, gamma, beta)


def kernel(global_data, segment_x, segment_edge_index, segment_edge_attr,
           segment_batch, dense_x, dense_edge_index, dense_batch,
           g0_W, g0_b, g0_gamma, g0_beta,
           g1_W, g1_b, g1_gamma, g1_beta,
           s0_Wl, s0_bl, s0_Wr, s0_br, s0_We, s0_be, s0_att, s0_bias,
           s0_gamma, s0_beta,
           s1_Wl, s1_bl, s1_Wr, s1_br, s1_We, s1_be, s1_att, s1_bias,
           s1_gamma, s1_beta,
           d0_Wl, d0_bl, d0_Wr, d0_br, d0_att, d0_bias, d0_gamma, d0_beta,
           d1_Wl, d1_bl, d1_Wr, d1_br, d1_att, d1_bias, d1_gamma, d1_beta,
           o_W, o_b):
    ssrc = segment_edge_index[0].astype(jnp.int32).reshape(_NW, _NCHUNK, _CH)
    sdst = segment_edge_index[1].astype(jnp.int32).reshape(_NW, _NCHUNK, _CH)
    dsrc = dense_edge_index[0].astype(jnp.int32).reshape(_NW, _NCHUNK, _CH)
    ddst = dense_edge_index[1].astype(jnp.int32).reshape(_NW, _NCHUNK, _CH)
    sb3 = segment_batch.astype(jnp.int32).reshape(_NRB, 1, _RB)
    db3 = dense_batch.astype(jnp.int32).reshape(_NRB, 1, _RB)

    g = _g_path(global_data, g0_W, g0_b, g0_gamma, g0_beta,
                g1_W, g1_b, g1_gamma, g1_beta)

    sx = segment_x
    sx = _gat_layer(sx, ssrc, sdst, s0_Wl, s0_bl, s0_Wr, s0_br, s0_att,
                    s0_bias, s0_gamma, s0_beta, s0_We, s0_be,
                    segment_edge_attr)
    sx = _gat_layer(sx, ssrc, sdst, s1_Wl, s1_bl, s1_Wr, s1_br, s1_att,
                    s1_bias, s1_gamma, s1_beta, s1_We, s1_be,
                    segment_edge_attr)

    dx = dense_x
    dx = _gat_layer(dx, dsrc, ddst, d0_Wl, d0_bl, d0_Wr, d0_br, d0_att,
                    d0_bias, d0_gamma, d0_beta)
    dx = _gat_layer(dx, dsrc, ddst, d1_Wl, d1_bl, d1_Wr, d1_br, d1_att,
                    d1_bias, d1_gamma, d1_beta)

    ow = jnp.transpose(o_W).reshape(1, 3, _H)
    return _pool_out(sx, dx, sb3, db3, g, ow, o_b)
